# baseline (speedup 1.0000x reference)
import jax
import jax.numpy as jnp
from jax.experimental import pallas as pl
from jax.experimental.pallas import tpu as pltpu

_P1_TN = 512
_P2_TM = 2048
_P2_TN = 128


def _mm1_kernel(x_ref, w1_ref, b1_ref, h_ref):
    w1b = w1_ref[...].astype(jnp.bfloat16)
    h = jnp.dot(x_ref[...], w1b, preferred_element_type=jnp.float32)
    h_ref[...] = jnp.maximum(h + b1_ref[...], 0.0).astype(jnp.bfloat16)


def _mm2_kernel(h_hbm, w2_ref, b2_ref, o_ref, h_vmem, sem):
    m = pl.program_id(0)

    @pl.when(pl.program_id(1) == 0)
    def _fetch():
        cp = pltpu.make_async_copy(
            h_hbm.at[pl.ds(m * _P2_TM, _P2_TM), :], h_vmem, sem
        )
        cp.start()
        cp.wait()

    w2b = w2_ref[...].astype(jnp.bfloat16)
    o_ref[...] = (
        jnp.dot(h_vmem[...], w2b, preferred_element_type=jnp.float32)
        + b2_ref[...]
    )


def kernel(x, gate, W1, b1, W2, b2):
    batch, cutoff, dmodel = x.shape
    n_tokens = batch * cutoff
    width = W1.shape[1]

    x2 = x.reshape(n_tokens, dmodel).astype(jnp.bfloat16)
    b1f = b1.astype(jnp.float32).reshape(1, width)
    b2f = b2.astype(jnp.float32).reshape(1, dmodel)

    h = pl.pallas_call(
        _mm1_kernel,
        grid=(width // _P1_TN,),
        in_specs=[
            pl.BlockSpec((n_tokens, dmodel), lambda n: (0, 0)),
            pl.BlockSpec((dmodel, _P1_TN), lambda n: (0, n)),
            pl.BlockSpec((1, _P1_TN), lambda n: (0, n)),
        ],
        out_specs=pl.BlockSpec((n_tokens, _P1_TN), lambda n: (0, n)),
        out_shape=jax.ShapeDtypeStruct((n_tokens, width), jnp.bfloat16),
        compiler_params=pltpu.CompilerParams(
            dimension_semantics=("arbitrary",),
            vmem_limit_bytes=128 * 1024 * 1024,
        ),
    )(x2, W1, b1f)

    out = pl.pallas_call(
        _mm2_kernel,
        grid=(n_tokens // _P2_TM, dmodel // _P2_TN),
        in_specs=[
            pl.BlockSpec(memory_space=pl.ANY),
            pl.BlockSpec((width, _P2_TN), lambda m, n: (0, n)),
            pl.BlockSpec((1, _P2_TN), lambda m, n: (0, n)),
        ],
        out_specs=pl.BlockSpec((_P2_TM, _P2_TN), lambda m, n: (m, n)),
        out_shape=jax.ShapeDtypeStruct((n_tokens, dmodel), jnp.float32),
        scratch_shapes=[
            pltpu.VMEM((_P2_TM, width), jnp.bfloat16),
            pltpu.SemaphoreType.DMA,
        ],
        compiler_params=pltpu.CompilerParams(
            dimension_semantics=("arbitrary", "arbitrary"),
            vmem_limit_bytes=128 * 1024 * 1024,
        ),
    )(h, W2, b2f)

    return out.reshape(batch, cutoff, dmodel)


# speedup vs baseline: 1.4841x; 1.4841x over previous
import jax
import jax.numpy as jnp
from jax.experimental import pallas as pl
from jax.experimental.pallas import tpu as pltpu

_P1_TN = 512
_P2_TM = 1024
_P2_TN = 512


def _mm1_kernel(x_ref, w1_ref, b1_ref, w2f_ref, h_ref, w2b_ref):
    w1b = w1_ref[...].astype(jnp.bfloat16)
    h = jnp.dot(x_ref[...], w1b, preferred_element_type=jnp.float32)
    h_ref[...] = jnp.maximum(h + b1_ref[...], 0.0).astype(jnp.bfloat16)
    w2b_ref[...] = w2f_ref[...].astype(jnp.bfloat16)


def _mm2_kernel(h_ref, w2_ref, b2_ref, o_ref):
    o_ref[...] = (
        jnp.dot(h_ref[...], w2_ref[...], preferred_element_type=jnp.float32)
        + b2_ref[...]
    )


def kernel(x, gate, W1, b1, W2, b2):
    batch, cutoff, dmodel = x.shape
    n_tokens = batch * cutoff
    width = W1.shape[1]

    x2 = x.reshape(n_tokens, dmodel).astype(jnp.bfloat16)
    b1f = b1.astype(jnp.float32).reshape(1, width)
    b2f = b2.astype(jnp.float32).reshape(1, dmodel)

    n1 = width // _P1_TN
    w2_rows = width // n1

    h, w2b = pl.pallas_call(
        _mm1_kernel,
        grid=(n1,),
        in_specs=[
            pl.BlockSpec((n_tokens, dmodel), lambda n: (0, 0)),
            pl.BlockSpec((dmodel, _P1_TN), lambda n: (0, n)),
            pl.BlockSpec((1, _P1_TN), lambda n: (0, n)),
            pl.BlockSpec((w2_rows, dmodel), lambda n: (n, 0)),
        ],
        out_specs=[
            pl.BlockSpec((n_tokens, _P1_TN), lambda n: (0, n)),
            pl.BlockSpec((w2_rows, dmodel), lambda n: (n, 0)),
        ],
        out_shape=[
            jax.ShapeDtypeStruct((n_tokens, width), jnp.bfloat16),
            jax.ShapeDtypeStruct((width, dmodel), jnp.bfloat16),
        ],
        compiler_params=pltpu.CompilerParams(
            dimension_semantics=("arbitrary",),
            vmem_limit_bytes=128 * 1024 * 1024,
        ),
    )(x2, W1, b1f, W2)

    out = pl.pallas_call(
        _mm2_kernel,
        grid=(n_tokens // _P2_TM, dmodel // _P2_TN),
        in_specs=[
            pl.BlockSpec((_P2_TM, width), lambda m, n: (m, 0)),
            pl.BlockSpec((width, _P2_TN), lambda m, n: (0, n)),
            pl.BlockSpec((1, _P2_TN), lambda m, n: (0, n)),
        ],
        out_specs=pl.BlockSpec((_P2_TM, _P2_TN), lambda m, n: (m, n)),
        out_shape=jax.ShapeDtypeStruct((n_tokens, dmodel), jnp.float32),
        compiler_params=pltpu.CompilerParams(
            dimension_semantics=("arbitrary", "arbitrary"),
            vmem_limit_bytes=128 * 1024 * 1024,
        ),
    )(h, w2b, b2f)

    return out.reshape(batch, cutoff, dmodel)
